# baseline (speedup 1.0000x reference)
import functools

import jax
import jax.numpy as jnp
from jax import lax
from jax.experimental import pallas as pl
from jax.experimental.pallas import tpu as pltpu
from jax.experimental.pallas import tpu_sc as plsc

VOCAB = 1000000
MAX_SEQ = 2048
DIM = 64
BATCH = 32

NUM_CORES = 2
NUM_SUBCORES = 16
NUM_WORKERS = NUM_CORES * NUM_SUBCORES
CHUNK = 512
NUM_CHUNKS = MAX_SEQ // CHUNK
LANES = 16
VECS_PER_ROW = DIM // LANES


def _sc_body(x_hbm, tok_hbm, pos_hbm, out_hbm, idx_v, rows_v, pos_v, sem):
    wid = lax.axis_index("s") * NUM_CORES + lax.axis_index("c")
    base = wid * MAX_SEQ

    pltpu.sync_copy(x_hbm.at[pl.ds(base, MAX_SEQ)], idx_v)

    for c in range(NUM_CHUNKS):
        gather = pltpu.async_copy(
            tok_hbm.at[idx_v.at[pl.ds(c * CHUNK, CHUNK)]], rows_v, sem)
        pltpu.sync_copy(pos_hbm.at[pl.ds(c * CHUNK, CHUNK)], pos_v)
        gather.wait()

        def add_row(i, _):
            for j in range(VECS_PER_ROW):
                s = pl.ds(j * LANES, LANES)
                rows_v[i, s] = rows_v[i, s] + pos_v[i, s]
            return 0

        lax.fori_loop(0, CHUNK, add_row, 0)

        pltpu.sync_copy(rows_v, out_hbm.at[pl.ds(base + c * CHUNK, CHUNK)])


@functools.partial(jax.jit, donate_argnums=())
def kernel(x, token_table, pos_table):
    x_flat = x.reshape(-1).astype(jnp.int32)
    mesh = plsc.VectorSubcoreMesh(core_axis_name="c", subcore_axis_name="s")
    out = pl.kernel(
        _sc_body,
        out_type=jax.ShapeDtypeStruct((BATCH * MAX_SEQ, DIM), jnp.float32),
        mesh=mesh,
        scratch_types=[
            pltpu.VMEM((MAX_SEQ,), jnp.int32),
            pltpu.VMEM((CHUNK, DIM), jnp.float32),
            pltpu.VMEM((CHUNK, DIM), jnp.float32),
            pltpu.SemaphoreType.DMA,
        ],
        compiler_params=pltpu.CompilerParams(use_tc_tiling_on_sc=False),
    )(x_flat, token_table, pos_table)
    return out.reshape(BATCH, MAX_SEQ, DIM)
